# baseline (speedup 1.0000x reference)
import functools

import jax
import jax.numpy as jnp
from jax import lax
from jax.experimental import pallas as pl
from jax.experimental.pallas import tpu as pltpu
from jax.experimental.pallas import tpu_sc as plsc

_NC = 2
_NS = 16
_NW = _NC * _NS
_CHUNK = 128


def _sc_mesh():
    return plsc.VectorSubcoreMesh(core_axis_name="c", subcore_axis_name="s")


def _sc_params():
    return pltpu.CompilerParams(use_tc_tiling_on_sc=False)


@functools.lru_cache(maxsize=None)
def _sc_count_kernel(n_chunks, n_rows):
    rows_per_tile = n_rows // _NS

    @functools.partial(
        pl.kernel,
        out_type=jax.ShapeDtypeStruct((_NC, n_rows, 16), jnp.float32),
        mesh=_sc_mesh(),
        compiler_params=_sc_params(),
        scratch_types=[
            pltpu.VMEM_SHARED((n_rows, 16), jnp.float32),
            pltpu.VMEM((n_chunks, _CHUNK), jnp.int32),
            pltpu.VMEM((_CHUNK, 16), jnp.float32),
        ],
    )
    def k(dst_hbm, ones_hbm, zeros_hbm, out_hbm, acc_sh, didx_v, ones_v):
        cid = lax.axis_index("c")
        sid = lax.axis_index("s")
        r0 = sid * rows_per_tile
        pltpu.sync_copy(zeros_hbm.at[pl.ds(r0, rows_per_tile)],
                        acc_sh.at[pl.ds(r0, rows_per_tile)])
        pltpu.sync_copy(ones_hbm, ones_v)
        pltpu.sync_copy(dst_hbm.at[sid], didx_v)
        plsc.subcore_barrier()
        n_mine = (n_chunks + 1 - cid) // 2

        def body(i, carry):
            pltpu.sync_copy(ones_v, acc_sh.at[didx_v.at[2 * i + cid]],
                            add=True)
            return carry

        lax.fori_loop(0, n_mine, body, 0)
        plsc.subcore_barrier()
        pltpu.sync_copy(acc_sh.at[pl.ds(r0, rows_per_tile)],
                        out_hbm.at[cid, pl.ds(r0, rows_per_tile)])

    return k


@functools.lru_cache(maxsize=None)
def _sc_segsum_kernel(n_chunks, n_rows, feat):
    rows_per_tile = n_rows // _NS
    fh = feat // _NC

    @functools.partial(
        pl.kernel,
        out_type=jax.ShapeDtypeStruct((n_rows, feat), jnp.float32),
        mesh=_sc_mesh(),
        compiler_params=_sc_params(),
        scratch_types=[
            pltpu.VMEM_SHARED((n_rows, fh), jnp.float32),
            pltpu.VMEM((n_chunks, _CHUNK), jnp.int32),
            pltpu.VMEM((n_chunks, _CHUNK), jnp.int32),
            pltpu.VMEM((_CHUNK, fh), jnp.float32),
            pltpu.VMEM((_CHUNK, fh), jnp.float32),
            pltpu.SemaphoreType.DMA,
            pltpu.SemaphoreType.DMA,
        ],
    )
    def k(hp_hbm, src_hbm, dst_hbm, zeros_hbm, out_hbm,
          acc_sh, sidx_v, didx_v, buf0, buf1, sem0, sem1):
        cid = lax.axis_index("c")
        sid = lax.axis_index("s")
        r0 = sid * rows_per_tile
        c0 = cid * fh
        ds_rows = pl.ds(r0, rows_per_tile)
        table = hp_hbm.at[cid]
        stage = [
            pltpu.make_async_copy(zeros_hbm.at[ds_rows],
                                  acc_sh.at[ds_rows], sem1),
            pltpu.make_async_copy(src_hbm.at[sid], sidx_v, sem0),
            pltpu.make_async_copy(dst_hbm.at[sid], didx_v, sem1),
        ]
        for d in stage:
            d.start()
        for d in stage:
            d.wait()
        plsc.subcore_barrier()

        pltpu.async_copy(table.at[sidx_v.at[0]], buf0, sem0)

        def pair(j, carry):
            cc = 2 * j
            pltpu.async_copy(table.at[sidx_v.at[cc + 1]], buf1, sem1)
            pltpu.make_async_copy(table.at[sidx_v.at[cc]], buf0, sem0).wait()
            pltpu.sync_copy(buf0, acc_sh.at[didx_v.at[cc]], add=True)
            pltpu.async_copy(table.at[sidx_v.at[cc + 2]], buf0, sem0)
            pltpu.make_async_copy(table.at[sidx_v.at[cc + 1]], buf1,
                                  sem1).wait()
            pltpu.sync_copy(buf1, acc_sh.at[didx_v.at[cc + 1]], add=True)
            return carry

        lax.fori_loop(0, (n_chunks - 1) // 2, pair, 0)
        last = n_chunks - 1
        pltpu.make_async_copy(table.at[sidx_v.at[last]], buf0, sem0).wait()
        pltpu.sync_copy(buf0, acc_sh.at[didx_v.at[last]], add=True)
        plsc.subcore_barrier()
        pltpu.sync_copy(acc_sh.at[pl.ds(r0, rows_per_tile)],
                        out_hbm.at[pl.ds(r0, rows_per_tile), pl.ds(c0, fh)])

    return k


def _tc_matmul(a, w):
    n = a.shape[0]
    f = w.shape[1]

    def body(a_ref, w_ref, h_ref):
        h_ref[...] = jnp.dot(a_ref[...], w_ref[...],
                             preferred_element_type=jnp.float32)

    return pl.pallas_call(
        body,
        out_shape=jax.ShapeDtypeStruct((n, f), jnp.float32),
    )(a, w)


def _write_hp(hp_ref, hp, n, n_rows, fh):
    pad = jnp.zeros((n_rows - n, fh), jnp.float32)
    hp_ref[0, :n] = hp[:, :fh]
    hp_ref[0, n:] = pad
    hp_ref[1, :n] = hp[:, fh:]
    hp_ref[1, n:] = pad


def _tc_deg_scale(cnt, h, n_rows):
    n, f = h.shape
    fh = f // _NC

    def body(c_ref, h_ref, inv_ref, invdeg_ref, hp_ref):
        deg = 1.0 + c_ref[0, :n, 0:1] + c_ref[1, :n, 0:1]
        inv = lax.rsqrt(deg)
        inv_ref[...] = inv
        invdeg_ref[...] = 1.0 / deg
        _write_hp(hp_ref, h_ref[...] * inv, n, n_rows, fh)

    return pl.pallas_call(
        body,
        out_shape=(jax.ShapeDtypeStruct((n, 1), jnp.float32),
                   jax.ShapeDtypeStruct((n, 1), jnp.float32),
                   jax.ShapeDtypeStruct((_NC, n_rows, fh), jnp.float32)),
    )(cnt, h)


def _bn_combine(p_ref, h_ref, inv_ref, invdeg_ref, b_ref, g_ref, be_ref,
                res_ref, n):
    e = p_ref[:n, :] * inv_ref[...]
    hh = h_ref[...]
    s = (e + hh * invdeg_ref[...]) * invdeg_ref[...] + b_ref[...]
    t = jnp.where(s >= 0.0, s, 0.01 * s)
    mu = jnp.mean(t, axis=0, keepdims=True)
    var = jnp.mean((t - mu) * (t - mu), axis=0, keepdims=True)
    return (res_ref[...]
            + (t - mu) * lax.rsqrt(var + 1e-5) * g_ref[...] + be_ref[...])


def _tc_fused(p, h, inv, invdeg, b, g, be, res, w_next, n_rows, want_a):
    n, f = h.shape
    f2 = w_next.shape[1]

    def body(p_ref, h_ref, inv_ref, invdeg_ref, b_ref, g_ref, be_ref,
             res_ref, w_ref, *out_refs):
        a = _bn_combine(p_ref, h_ref, inv_ref, invdeg_ref, b_ref, g_ref,
                        be_ref, res_ref, n)
        if want_a:
            out_refs[0][...] = a
        h2_ref, hp2_ref = out_refs[-2], out_refs[-1]
        h2 = jnp.dot(a, w_ref[...], preferred_element_type=jnp.float32)
        h2_ref[...] = h2
        _write_hp(hp2_ref, h2 * inv_ref[...], n, n_rows, f2 // _NC)

    shapes = (jax.ShapeDtypeStruct((n, f2), jnp.float32),
              jax.ShapeDtypeStruct((_NC, n_rows, f2 // _NC), jnp.float32))
    if want_a:
        shapes = (jax.ShapeDtypeStruct((n, f), jnp.float32),) + shapes
    return pl.pallas_call(
        body,
        out_shape=shapes,
    )(p, h, inv, invdeg, b.reshape(1, f), g.reshape(1, f), be.reshape(1, f),
      res, w_next)


def _tc_tail(p, h, inv, invdeg, b, g, be, res, wl, bl):
    n, f = h.shape
    f2 = wl.shape[1]

    def body(p_ref, h_ref, inv_ref, invdeg_ref, b_ref, g_ref, be_ref,
             res_ref, w_ref, bl_ref, y_ref):
        a = _bn_combine(p_ref, h_ref, inv_ref, invdeg_ref, b_ref, g_ref,
                        be_ref, res_ref, n)
        y_ref[...] = (jnp.dot(a, w_ref[...], preferred_element_type=jnp.float32)
                      + bl_ref[...])

    return pl.pallas_call(
        body,
        out_shape=jax.ShapeDtypeStruct((n, f2), jnp.float32),
    )(p, h, inv, invdeg, b.reshape(1, f), g.reshape(1, f), be.reshape(1, f),
      res, wl, bl.reshape(1, f2))


def kernel(x, edge_index, ptr, params):
    n = x.shape[0]
    n_rows = ((n // (_NS * 8)) + 1) * (_NS * 8)
    e = edge_index.shape[1]
    batch = int(ptr.shape[0]) - 1
    out_ch = params['Wl'].shape[1]

    grain = _NS * _CHUNK
    n_chunks = max(1, (e + grain - 1) // grain)
    if n_chunks % 2 == 0:
        n_chunks += 1
    epad = n_chunks * grain
    pad_cfg = ((0, epad - e),)
    idx_shape = (_NS, n_chunks, _CHUNK)
    src = jnp.pad(edge_index[0], pad_cfg, constant_values=n).reshape(idx_shape)
    dst = jnp.pad(edge_index[1], pad_cfg, constant_values=n).reshape(idx_shape)

    ones16 = jnp.ones((_CHUNK, 16), jnp.float32)
    zeros16 = jnp.zeros((n_rows, 16), jnp.float32)

    cnt = _sc_count_kernel(n_chunks, n_rows)(dst, ones16, zeros16)
    hm = _tc_matmul(x, params['W0'])
    inv, invdeg, hp = _tc_deg_scale(cnt, hm, n_rows)

    feats = []
    for i in range(8):
        f = params['W%d' % i].shape[1]
        zeros_f = jnp.zeros((n_rows, f // _NC), jnp.float32)
        p = _sc_segsum_kernel(n_chunks, n_rows, f)(hp, src, dst, zeros_f)
        res = feats[7 - i] if i >= 4 else jnp.zeros((n, f), jnp.float32)
        bgb = (params['b%d' % i], params['g%d' % i], params['be%d' % i])
        if i < 7:
            outs = _tc_fused(p, hm, inv, invdeg, *bgb, res,
                             params['W%d' % (i + 1)], n_rows, i < 4)
            if i < 4:
                feats.append(outs[0])
            hm, hp = outs[-2], outs[-1]
        else:
            y = _tc_tail(p, hm, inv, invdeg, *bgb, res,
                         params['Wl'], params['bl'])

    y = y.reshape(batch, n // batch, out_ch)
    return jnp.transpose(y, (0, 2, 1))


# speedup vs baseline: 1.3749x; 1.3749x over previous
import functools

import jax
import jax.numpy as jnp
from jax import lax
from jax.experimental import pallas as pl
from jax.experimental.pallas import tpu as pltpu
from jax.experimental.pallas import tpu_sc as plsc

_NC = 2
_NS = 16
_NW = _NC * _NS
_CHUNK = 128


def _sc_mesh():
    return plsc.VectorSubcoreMesh(core_axis_name="c", subcore_axis_name="s")


def _sc_params():
    return pltpu.CompilerParams(use_tc_tiling_on_sc=False)


@functools.lru_cache(maxsize=None)
def _sc_count_kernel(n_chunks, n_rows):
    rows_per_tile = n_rows // _NS

    @functools.partial(
        pl.kernel,
        out_type=jax.ShapeDtypeStruct((_NC, n_rows, 16), jnp.float32),
        mesh=_sc_mesh(),
        compiler_params=_sc_params(),
        scratch_types=[
            pltpu.VMEM_SHARED((n_rows, 16), jnp.float32),
            pltpu.VMEM((n_chunks, _CHUNK), jnp.int32),
            pltpu.VMEM((_CHUNK, 16), jnp.float32),
        ],
    )
    def k(dst_hbm, ones_hbm, zeros_hbm, out_hbm, acc_sh, didx_v, ones_v):
        cid = lax.axis_index("c")
        sid = lax.axis_index("s")
        r0 = sid * rows_per_tile
        pltpu.sync_copy(zeros_hbm.at[pl.ds(r0, rows_per_tile)],
                        acc_sh.at[pl.ds(r0, rows_per_tile)])
        pltpu.sync_copy(ones_hbm, ones_v)
        pltpu.sync_copy(dst_hbm.at[sid], didx_v)
        plsc.subcore_barrier()
        n_mine = (n_chunks + 1 - cid) // 2

        def body(i, carry):
            pltpu.sync_copy(ones_v, acc_sh.at[didx_v.at[2 * i + cid]],
                            add=True)
            return carry

        lax.fori_loop(0, n_mine, body, 0)
        plsc.subcore_barrier()
        pltpu.sync_copy(acc_sh.at[pl.ds(r0, rows_per_tile)],
                        out_hbm.at[cid, pl.ds(r0, rows_per_tile)])

    return k


@functools.lru_cache(maxsize=None)
def _sc_segsum_kernel(n_chunks, n_rows, feat):
    rows_per_tile = n_rows // _NS
    fh = feat // _NC

    @functools.partial(
        pl.kernel,
        out_type=jax.ShapeDtypeStruct((n_rows, feat), jnp.float32),
        mesh=_sc_mesh(),
        compiler_params=_sc_params(),
        scratch_types=[
            pltpu.VMEM_SHARED((n_rows, fh), jnp.float32),
            pltpu.VMEM_SHARED((n_rows, fh), jnp.float32),
            pltpu.VMEM((n_chunks, _CHUNK), jnp.int32),
            pltpu.VMEM((n_chunks, _CHUNK), jnp.int32),
            pltpu.VMEM((_CHUNK, fh), jnp.float32),
            pltpu.VMEM((_CHUNK, fh), jnp.float32),
            pltpu.SemaphoreType.DMA,
            pltpu.SemaphoreType.DMA,
        ],
    )
    def k(hp_hbm, src_hbm, dst_hbm, zeros_hbm, out_hbm,
          table_sh, acc_sh, sidx_v, didx_v, buf0, buf1, sem0, sem1):
        cid = lax.axis_index("c")
        sid = lax.axis_index("s")
        r0 = sid * rows_per_tile
        c0 = cid * fh
        ds_rows = pl.ds(r0, rows_per_tile)
        stage = [
            pltpu.make_async_copy(hp_hbm.at[ds_rows, pl.ds(c0, fh)],
                                  table_sh.at[ds_rows], sem0),
            pltpu.make_async_copy(zeros_hbm.at[ds_rows],
                                  acc_sh.at[ds_rows], sem1),
            pltpu.make_async_copy(src_hbm.at[sid], sidx_v, sem0),
            pltpu.make_async_copy(dst_hbm.at[sid], didx_v, sem1),
        ]
        for d in stage:
            d.start()
        for d in stage:
            d.wait()
        plsc.subcore_barrier()

        pltpu.async_copy(table_sh.at[sidx_v.at[0]], buf0, sem0)

        def pair(j, carry):
            c0 = 2 * j
            pltpu.async_copy(table_sh.at[sidx_v.at[c0 + 1]], buf1, sem1)
            pltpu.make_async_copy(table_sh.at[sidx_v.at[c0]], buf0, sem0).wait()
            pltpu.sync_copy(buf0, acc_sh.at[didx_v.at[c0]], add=True)
            pltpu.async_copy(table_sh.at[sidx_v.at[c0 + 2]], buf0, sem0)
            pltpu.make_async_copy(table_sh.at[sidx_v.at[c0 + 1]], buf1,
                                  sem1).wait()
            pltpu.sync_copy(buf1, acc_sh.at[didx_v.at[c0 + 1]], add=True)
            return carry

        lax.fori_loop(0, (n_chunks - 1) // 2, pair, 0)
        last = n_chunks - 1
        pltpu.make_async_copy(table_sh.at[sidx_v.at[last]], buf0, sem0).wait()
        pltpu.sync_copy(buf0, acc_sh.at[didx_v.at[last]], add=True)
        plsc.subcore_barrier()
        pltpu.sync_copy(acc_sh.at[pl.ds(r0, rows_per_tile)],
                        out_hbm.at[pl.ds(r0, rows_per_tile), pl.ds(c0, fh)])

    return k


def _tc_matmul(a, w):
    n = a.shape[0]
    f = w.shape[1]

    def body(a_ref, w_ref, h_ref):
        h_ref[...] = jnp.dot(a_ref[...], w_ref[...],
                             preferred_element_type=jnp.float32)

    return pl.pallas_call(
        body,
        out_shape=jax.ShapeDtypeStruct((n, f), jnp.float32),
    )(a, w)


def _tc_deg_scale(cnt, h, n_rows):
    n, f = h.shape

    def body(c_ref, h_ref, inv_ref, invdeg_ref, hp_ref):
        deg = 1.0 + c_ref[0, :n, 0:1] + c_ref[1, :n, 0:1]
        inv = lax.rsqrt(deg)
        inv_ref[...] = inv
        invdeg_ref[...] = 1.0 / deg
        hp_ref[:n] = h_ref[...] * inv
        hp_ref[n:] = jnp.zeros((n_rows - n, f), jnp.float32)

    return pl.pallas_call(
        body,
        out_shape=(jax.ShapeDtypeStruct((n, 1), jnp.float32),
                   jax.ShapeDtypeStruct((n, 1), jnp.float32),
                   jax.ShapeDtypeStruct((n_rows, f), jnp.float32)),
    )(cnt, h)


def _bn_combine(p_ref, h_ref, inv_ref, invdeg_ref, b_ref, g_ref, be_ref,
                res_ref, n):
    e = p_ref[:n, :] * inv_ref[...]
    hh = h_ref[...]
    s = (e + hh * invdeg_ref[...]) * invdeg_ref[...] + b_ref[...]
    t = jnp.where(s >= 0.0, s, 0.01 * s)
    mu = jnp.mean(t, axis=0, keepdims=True)
    var = jnp.mean((t - mu) * (t - mu), axis=0, keepdims=True)
    return (res_ref[...]
            + (t - mu) * lax.rsqrt(var + 1e-5) * g_ref[...] + be_ref[...])


def _tc_fused(p, h, inv, invdeg, b, g, be, res, w_next, n_rows, want_a):
    n, f = h.shape
    f2 = w_next.shape[1]

    def body(p_ref, h_ref, inv_ref, invdeg_ref, b_ref, g_ref, be_ref,
             res_ref, w_ref, *out_refs):
        a = _bn_combine(p_ref, h_ref, inv_ref, invdeg_ref, b_ref, g_ref,
                        be_ref, res_ref, n)
        if want_a:
            out_refs[0][...] = a
        h2_ref, hp2_ref = out_refs[-2], out_refs[-1]
        h2 = jnp.dot(a, w_ref[...], preferred_element_type=jnp.float32)
        h2_ref[...] = h2
        hp2_ref[:n] = h2 * inv_ref[...]
        hp2_ref[n:] = jnp.zeros((n_rows - n, f2), jnp.float32)

    shapes = (jax.ShapeDtypeStruct((n, f2), jnp.float32),
              jax.ShapeDtypeStruct((n_rows, f2), jnp.float32))
    if want_a:
        shapes = (jax.ShapeDtypeStruct((n, f), jnp.float32),) + shapes
    return pl.pallas_call(
        body,
        out_shape=shapes,
    )(p, h, inv, invdeg, b.reshape(1, f), g.reshape(1, f), be.reshape(1, f),
      res, w_next)


def _tc_tail(p, h, inv, invdeg, b, g, be, res, wl, bl):
    n, f = h.shape
    f2 = wl.shape[1]

    def body(p_ref, h_ref, inv_ref, invdeg_ref, b_ref, g_ref, be_ref,
             res_ref, w_ref, bl_ref, y_ref):
        a = _bn_combine(p_ref, h_ref, inv_ref, invdeg_ref, b_ref, g_ref,
                        be_ref, res_ref, n)
        y_ref[...] = (jnp.dot(a, w_ref[...], preferred_element_type=jnp.float32)
                      + bl_ref[...])

    return pl.pallas_call(
        body,
        out_shape=jax.ShapeDtypeStruct((n, f2), jnp.float32),
    )(p, h, inv, invdeg, b.reshape(1, f), g.reshape(1, f), be.reshape(1, f),
      res, wl, bl.reshape(1, f2))


def kernel(x, edge_index, ptr, params):
    n = x.shape[0]
    n_rows = ((n // (_NS * 8)) + 1) * (_NS * 8)
    e = edge_index.shape[1]
    batch = int(ptr.shape[0]) - 1
    out_ch = params['Wl'].shape[1]

    grain = _NS * _CHUNK
    n_chunks = max(1, (e + grain - 1) // grain)
    if n_chunks % 2 == 0:
        n_chunks += 1
    epad = n_chunks * grain
    pad_cfg = ((0, epad - e),)
    idx_shape = (_NS, n_chunks, _CHUNK)
    src = jnp.pad(edge_index[0], pad_cfg, constant_values=n).reshape(idx_shape)
    dst = jnp.pad(edge_index[1], pad_cfg, constant_values=n).reshape(idx_shape)

    ones16 = jnp.ones((_CHUNK, 16), jnp.float32)
    zeros16 = jnp.zeros((n_rows, 16), jnp.float32)

    cnt = _sc_count_kernel(n_chunks, n_rows)(dst, ones16, zeros16)
    hm = _tc_matmul(x, params['W0'])
    inv, invdeg, hp = _tc_deg_scale(cnt, hm, n_rows)

    feats = []
    for i in range(8):
        f = params['W%d' % i].shape[1]
        zeros_f = jnp.zeros((n_rows, f // _NC), jnp.float32)
        p = _sc_segsum_kernel(n_chunks, n_rows, f)(hp, src, dst, zeros_f)
        res = feats[7 - i] if i >= 4 else jnp.zeros((n, f), jnp.float32)
        bgb = (params['b%d' % i], params['g%d' % i], params['be%d' % i])
        if i < 7:
            outs = _tc_fused(p, hm, inv, invdeg, *bgb, res,
                             params['W%d' % (i + 1)], n_rows, i < 4)
            if i < 4:
                feats.append(outs[0])
            hm, hp = outs[-2], outs[-1]
        else:
            y = _tc_tail(p, hm, inv, invdeg, *bgb, res,
                         params['Wl'], params['bl'])

    y = y.reshape(batch, n // batch, out_ch)
    return jnp.transpose(y, (0, 2, 1))
